# Initial kernel scaffold; baseline (speedup 1.0000x reference)
import jax
import jax.numpy as jnp
from jax.experimental import pallas as pl


def kernel(x, table):
    raise NotImplementedError("write your pallas kernel here")


# speedup vs baseline: 2.6418x; 2.6418x over previous
import functools

import jax
import jax.numpy as jnp
from jax import lax
from jax.experimental import pallas as pl
from jax.experimental.pallas import tpu as pltpu
from jax.experimental.pallas import tpu_sc as plsc

NC, NS, L = 2, 16, 16
NW = NC * NS
BT = 128
K = 6
P = 1024
E = 64
V = 1000
Q = 256
ITEMS = BT * (P // Q)
IPW = ITEMS // NW


def _sc_embed(x2, table):
    mesh = plsc.VectorSubcoreMesh(
        core_axis_name="c", subcore_axis_name="s",
        num_cores=NC, num_subcores=NS)

    @functools.partial(
        pl.kernel,
        out_type=jax.ShapeDtypeStruct((BT, E, P), jnp.float32),
        mesh=mesh,
        compiler_params=pltpu.CompilerParams(needs_layout_passes=False),
        scratch_types=[
            pltpu.VMEM((V * E,), jnp.float32),
            pltpu.VMEM((K, Q), jnp.int32),
            pltpu.VMEM((E, Q), jnp.float32),
        ],
    )
    def k(x_hbm, table_hbm, out_hbm, table_v, idx_v, out_v):
        c = lax.axis_index("c")
        s = lax.axis_index("s")
        wid = s * NC + c
        pltpu.sync_copy(table_hbm, table_v)

        def item_body(it, carry):
            g = wid * IPW + it
            bt = g // (P // Q)
            p0 = (g % (P // Q)) * Q
            pltpu.sync_copy(x_hbm.at[bt, :, pl.ds(p0, Q)], idx_v)

            def pv_body(pv, carry2):
                rows = [idx_v[kk, pl.ds(pv * L, L)] * E for kk in range(K)]
                for e in range(E):
                    acc = plsc.load_gather(table_v, [rows[0] + e])
                    for kk in range(1, K):
                        acc = acc + plsc.load_gather(table_v, [rows[kk] + e])
                    out_v[e, pl.ds(pv * L, L)] = acc
                return carry2

            lax.fori_loop(0, Q // L, pv_body, 0)
            pltpu.sync_copy(out_v, out_hbm.at[bt, :, pl.ds(p0, Q)])
            return carry

        lax.fori_loop(0, IPW, item_body, 0)

    return k(x2, table.reshape(V * E))


def kernel(x, table):
    x2 = x.reshape(BT, K, P)
    out = _sc_embed(x2, table)
    return out.reshape(16, 8, E, 32, 32)


# speedup vs baseline: 9.6967x; 3.6705x over previous
import functools

import jax
import jax.numpy as jnp
from jax import lax
from jax.experimental import pallas as pl
from jax.experimental.pallas import tpu as pltpu
from jax.experimental.pallas import tpu_sc as plsc

NC, NS, L = 2, 16, 16
NW = NC * NS
BT = 128
K = 6
P = 1024
E = 64
V = 1000
Q = 256
ITEMS = BT * (P // Q)
IPW = ITEMS // NW


def _sc_embed(x2, table):
    mesh = plsc.VectorSubcoreMesh(
        core_axis_name="c", subcore_axis_name="s",
        num_cores=NC, num_subcores=NS)

    @functools.partial(
        pl.kernel,
        out_type=jax.ShapeDtypeStruct((BT, E, P), jnp.float32),
        mesh=mesh,
        compiler_params=pltpu.CompilerParams(needs_layout_passes=False),
        scratch_types=[
            pltpu.VMEM((V * (E + 1),), jnp.float32),
            pltpu.VMEM((K, Q), jnp.int32),
            pltpu.VMEM((E, Q), jnp.float32),
        ],
    )
    def k(x_hbm, table_hbm, out_hbm, table_v, idx_v, out_v):
        c = lax.axis_index("c")
        s = lax.axis_index("s")
        wid = s * NC + c
        pltpu.sync_copy(table_hbm, table_v)

        def item_body(it, carry):
            g = wid * IPW + it
            bt = g // (P // Q)
            p0 = (g % (P // Q)) * Q
            pltpu.sync_copy(x_hbm.at[bt, :, pl.ds(p0, Q)], idx_v)

            def pv_body(pv, carry2):
                rows = [idx_v[kk, pl.ds(pv * L, L)] * (E + 1) for kk in range(K)]
                for e in range(E):
                    g = [plsc.load_gather(table_v, [rows[kk] + e])
                         for kk in range(K)]
                    acc = ((g[0] + g[1]) + (g[2] + g[3])) + (g[4] + g[5])
                    out_v[e, pl.ds(pv * L, L)] = acc
                return carry2

            lax.fori_loop(0, Q // L, pv_body, 0)
            pltpu.sync_copy(out_v, out_hbm.at[bt, :, pl.ds(p0, Q)])
            return carry

        lax.fori_loop(0, IPW, item_body, 0)

    return k(x2, table)


def kernel(x, table):
    x2 = x.reshape(BT, K, P)
    tpad = jnp.pad(table, ((0, 0), (0, 1))).reshape(V * (E + 1))
    out = _sc_embed(x2, tpad)
    return out.reshape(16, 8, E, 32, 32)


# speedup vs baseline: 13.5531x; 1.3977x over previous
import functools

import jax
import jax.numpy as jnp
from jax import lax
from jax.experimental import pallas as pl
from jax.experimental.pallas import tpu as pltpu
from jax.experimental.pallas import tpu_sc as plsc

NC, NS, L = 2, 16, 16
NW = NC * NS
BT = 128
K = 6
P = 1024
E = 64
V = 1000
Q = 256
ITEMS = BT * (P // Q)
IPW = ITEMS // NW


def _sc_embed(x2, table):
    mesh = plsc.VectorSubcoreMesh(
        core_axis_name="c", subcore_axis_name="s",
        num_cores=NC, num_subcores=NS)

    @functools.partial(
        pl.kernel,
        out_type=jax.ShapeDtypeStruct((BT, E, P), jnp.float32),
        mesh=mesh,
        compiler_params=pltpu.CompilerParams(needs_layout_passes=False),
        scratch_types=[
            pltpu.VMEM((V * (E + 1),), jnp.float32),
            pltpu.VMEM((K, Q), jnp.int32),
            pltpu.VMEM((E, Q), jnp.float32),
        ],
    )
    def k(x_hbm, table_hbm, out_hbm, table_v, idx_v, out_v):
        c = lax.axis_index("c")
        s = lax.axis_index("s")
        wid = s * NC + c
        pltpu.sync_copy(table_hbm, table_v)

        def item_body(it, carry):
            g = wid * IPW + it
            bt = g // (P // Q)
            p0 = (g % (P // Q)) * Q
            pltpu.sync_copy(x_hbm.at[bt, :, pl.ds(p0, Q)], idx_v)

            def pv_body(pv, carry2):
                rows = [idx_v[kk, pl.ds(pv * L, L)] * (E + 1) for kk in range(K)]

                def gathers(e):
                    return [plsc.load_gather(table_v, [rows[kk] + e])
                            for kk in range(K)]

                g = gathers(0)
                for e in range(1, E):
                    ng = gathers(e)
                    acc = ((g[0] + g[1]) + (g[2] + g[3])) + (g[4] + g[5])
                    out_v[e - 1, pl.ds(pv * L, L)] = acc
                    g = ng
                acc = ((g[0] + g[1]) + (g[2] + g[3])) + (g[4] + g[5])
                out_v[E - 1, pl.ds(pv * L, L)] = acc
                return carry2

            lax.fori_loop(0, Q // L, pv_body, 0)
            pltpu.sync_copy(out_v, out_hbm.at[bt, :, pl.ds(p0, Q)])
            return carry

        lax.fori_loop(0, IPW, item_body, 0)

    return k(x2, table)


def kernel(x, table):
    x2 = x.reshape(BT, K, P)
    tpad = jnp.pad(table, ((0, 0), (0, 1))).reshape(V * (E + 1))
    out = _sc_embed(x2, tpad)
    return out.reshape(16, 8, E, 32, 32)


# speedup vs baseline: 17.4079x; 1.2844x over previous
import functools

import jax
import jax.numpy as jnp
from jax import lax
from jax.experimental import pallas as pl
from jax.experimental.pallas import tpu as pltpu
from jax.experimental.pallas import tpu_sc as plsc

NC, NS, L = 2, 16, 16
NW = NC * NS
BT = 128
K = 6
P = 1024
E = 64
EP = E // 2
W = EP + 1
V = 1000
Q = 256
ITEMS = BT * (P // Q)
IPW = ITEMS // NW


def _sc_embed(x2, tpack):
    mesh = plsc.VectorSubcoreMesh(
        core_axis_name="c", subcore_axis_name="s",
        num_cores=NC, num_subcores=NS)

    @functools.partial(
        pl.kernel,
        out_type=jax.ShapeDtypeStruct((BT, E, P), jnp.float32),
        mesh=mesh,
        compiler_params=pltpu.CompilerParams(needs_layout_passes=False),
        scratch_types=[
            pltpu.VMEM((V * W,), jnp.int32),
            pltpu.VMEM((K, Q), jnp.int32),
            pltpu.VMEM((E, Q), jnp.float32),
        ],
    )
    def k(x_hbm, table_hbm, out_hbm, table_v, idx_v, out_v):
        c = lax.axis_index("c")
        s = lax.axis_index("s")
        wid = s * NC + c
        pltpu.sync_copy(table_hbm, table_v)

        def item_body(it, carry):
            g = wid * IPW + it
            bt = g // (P // Q)
            p0 = (g % (P // Q)) * Q
            pltpu.sync_copy(x_hbm.at[bt, :, pl.ds(p0, Q)], idx_v)

            def pv_body(pv, carry2):
                rows = [idx_v[kk, pl.ds(pv * L, L)] * W for kk in range(K)]

                def gathers(ep):
                    return [plsc.load_gather(table_v, [rows[kk] + ep])
                            for kk in range(K)]

                def reduce_store(g, ep):
                    b = [plsc.bitcast(gi, jnp.bfloat16) for gi in g]
                    acc = (((b[0] + b[1]) + (b[2] + b[3]))
                           + (b[4] + b[5]))
                    lo, hi = plsc.unpack(
                        acc, format=plsc.PackFormat.INTERLEAVED)
                    out_v[2 * ep, pl.ds(pv * L, L)] = lo
                    out_v[2 * ep + 1, pl.ds(pv * L, L)] = hi

                g = gathers(0)
                for ep in range(1, EP):
                    ng = gathers(ep)
                    reduce_store(g, ep - 1)
                    g = ng
                reduce_store(g, EP - 1)
                return carry2

            lax.fori_loop(0, Q // L, pv_body, 0)
            pltpu.sync_copy(out_v, out_hbm.at[bt, :, pl.ds(p0, Q)])
            return carry

        lax.fori_loop(0, IPW, item_body, 0)

    return k(x2, tpack)


def kernel(x, table):
    x2 = x.reshape(BT, K, P)
    tb = jnp.pad(table.astype(jnp.bfloat16), ((0, 0), (0, 2)))
    tpack = jax.lax.bitcast_convert_type(
        tb.reshape(V, W, 2), jnp.int32).reshape(V * W)
    out = _sc_embed(x2, tpack)
    return out.reshape(16, 8, E, 32, 32)


# speedup vs baseline: 18.1671x; 1.0436x over previous
import functools

import jax
import jax.numpy as jnp
from jax import lax
from jax.experimental import pallas as pl
from jax.experimental.pallas import tpu as pltpu
from jax.experimental.pallas import tpu_sc as plsc

NC, NS, L = 2, 16, 16
NW = NC * NS
BT = 128
K = 6
P = 1024
E = 64
EP = E // 2
W = EP
V = 1000
Q = 256
ITEMS = BT * (P // Q)
IPW = ITEMS // NW


def _sc_embed(x2, tpack):
    mesh = plsc.VectorSubcoreMesh(
        core_axis_name="c", subcore_axis_name="s",
        num_cores=NC, num_subcores=NS)

    @functools.partial(
        pl.kernel,
        out_type=jax.ShapeDtypeStruct((BT, E, P), jnp.float32),
        mesh=mesh,
        compiler_params=pltpu.CompilerParams(needs_layout_passes=False),
        scratch_types=[
            pltpu.VMEM((V * W,), jnp.int32),
            pltpu.VMEM((K, Q), jnp.int32),
            pltpu.VMEM((E, Q), jnp.float32),
        ],
    )
    def k(x_hbm, table_hbm, out_hbm, table_v, idx_v, out_v):
        c = lax.axis_index("c")
        s = lax.axis_index("s")
        wid = s * NC + c
        pltpu.sync_copy(table_hbm, table_v)

        def item_body(it, carry):
            g = wid * IPW + it
            bt = g // (P // Q)
            p0 = (g % (P // Q)) * Q
            pltpu.sync_copy(x_hbm.at[bt, :, pl.ds(p0, Q)], idx_v)

            iota = lax.broadcasted_iota(jnp.int32, (L,), 0)

            def pv_body(pv, carry2):
                rows = [idx_v[kk, pl.ds(pv * L, L)] * W for kk in range(K)]
                p_idx = pv * L + iota

                def gathers(group):
                    j, h = group & 15, group >> 4
                    ep = ((iota + j) & 15) + 16 * h
                    return ep, [plsc.load_gather(table_v, [rows[kk] + ep])
                                for kk in range(K)]

                def reduce_scatter(ep, g):
                    b = [plsc.bitcast(gi, jnp.bfloat16) for gi in g]
                    acc = (((b[0] + b[1]) + (b[2] + b[3]))
                           + (b[4] + b[5]))
                    lo, hi = plsc.unpack(
                        acc, format=plsc.PackFormat.INTERLEAVED)
                    e_lo = 2 * ep
                    plsc.store_scatter(out_v, [e_lo, p_idx], lo)
                    plsc.store_scatter(out_v, [e_lo + 1, p_idx], hi)

                ep, g = gathers(0)
                for group in range(1, EP):
                    nep, ng = gathers(group)
                    reduce_scatter(ep, g)
                    ep, g = nep, ng
                reduce_scatter(ep, g)
                return carry2

            lax.fori_loop(0, Q // L, pv_body, 0)
            pltpu.sync_copy(out_v, out_hbm.at[bt, :, pl.ds(p0, Q)])
            return carry

        lax.fori_loop(0, IPW, item_body, 0)

    return k(x2, tpack)


def kernel(x, table):
    x2 = x.reshape(BT, K, P)
    tpack = jax.lax.bitcast_convert_type(
        table.astype(jnp.bfloat16).reshape(V, W, 2), jnp.int32).reshape(V * W)
    out = _sc_embed(x2, tpack)
    return out.reshape(16, 8, E, 32, 32)


# speedup vs baseline: 18.8393x; 1.0370x over previous
import functools

import jax
import jax.numpy as jnp
from jax import lax
from jax.experimental import pallas as pl
from jax.experimental.pallas import tpu as pltpu
from jax.experimental.pallas import tpu_sc as plsc

NC, NS, L = 2, 16, 16
NW = NC * NS
BT = 128
K = 6
P = 1024
E = 64
EP = E // 2
W = EP
V = 1000
Q = 512
ITEMS = BT * (P // Q)
IPW = ITEMS // NW


def _sc_embed(x2, tpack):
    mesh = plsc.VectorSubcoreMesh(
        core_axis_name="c", subcore_axis_name="s",
        num_cores=NC, num_subcores=NS)

    @functools.partial(
        pl.kernel,
        out_type=jax.ShapeDtypeStruct((BT, E, P), jnp.float32),
        mesh=mesh,
        compiler_params=pltpu.CompilerParams(needs_layout_passes=False),
        scratch_types=[
            pltpu.VMEM((V * W,), jnp.int32),
            pltpu.VMEM((K, Q), jnp.int32),
            pltpu.VMEM((E, Q), jnp.float32),
        ],
    )
    def k(x_hbm, table_hbm, out_hbm, table_v, idx_v, out_v):
        c = lax.axis_index("c")
        s = lax.axis_index("s")
        wid = s * NC + c
        pltpu.sync_copy(table_hbm, table_v)

        def item_body(it, carry):
            g = wid * IPW + it
            bt = g // (P // Q)
            p0 = (g % (P // Q)) * Q
            pltpu.sync_copy(x_hbm.at[bt, :, pl.ds(p0, Q)], idx_v)

            iota = lax.broadcasted_iota(jnp.int32, (L,), 0)

            def pv_body(pv, carry2):
                rows = [idx_v[kk, pl.ds(pv * L, L)] * W for kk in range(K)]
                p_idx = pv * L + iota

                def gathers(group):
                    j, h = group & 15, group >> 4
                    ep = ((iota + j) & 15) + 16 * h
                    return ep, [plsc.load_gather(table_v, [rows[kk] + ep])
                                for kk in range(K)]

                def reduce_scatter(ep, g):
                    b = [plsc.bitcast(gi, jnp.bfloat16) for gi in g]
                    acc = (((b[0] + b[1]) + (b[2] + b[3]))
                           + (b[4] + b[5]))
                    lo, hi = plsc.unpack(
                        acc, format=plsc.PackFormat.INTERLEAVED)
                    e_lo = 2 * ep
                    plsc.store_scatter(out_v, [e_lo, p_idx], lo)
                    plsc.store_scatter(out_v, [e_lo + 1, p_idx], hi)

                ep, g = gathers(0)
                for group in range(1, EP):
                    nep, ng = gathers(group)
                    reduce_scatter(ep, g)
                    ep, g = nep, ng
                reduce_scatter(ep, g)
                return carry2

            lax.fori_loop(0, Q // L, pv_body, 0)
            pltpu.sync_copy(out_v, out_hbm.at[bt, :, pl.ds(p0, Q)])
            return carry

        lax.fori_loop(0, IPW, item_body, 0)

    return k(x2, tpack)


def kernel(x, table):
    x2 = x.reshape(BT, K, P)
    tpack = jax.lax.bitcast_convert_type(
        table.astype(jnp.bfloat16).reshape(V, W, 2), jnp.int32).reshape(V * W)
    out = _sc_embed(x2, tpack)
    return out.reshape(16, 8, E, 32, 32)


# speedup vs baseline: 20.6014x; 1.0935x over previous
import functools

import jax
import jax.numpy as jnp
from jax import lax
from jax.experimental import pallas as pl
from jax.experimental.pallas import tpu as pltpu
from jax.experimental.pallas import tpu_sc as plsc

NC, NS, L = 2, 16, 16
NW = NC * NS
BT = 128
K = 6
P = 1024
E = 64
EP = E // 2
W = EP
V = 1000
Q = 512
ITEMS = BT * (P // Q)
IPW = ITEMS // NW


def _sc_embed(x2, tpack):
    mesh = plsc.VectorSubcoreMesh(
        core_axis_name="c", subcore_axis_name="s",
        num_cores=NC, num_subcores=NS)

    @functools.partial(
        pl.kernel,
        out_type=jax.ShapeDtypeStruct((BT, E, P), jnp.float32),
        mesh=mesh,
        compiler_params=pltpu.CompilerParams(needs_layout_passes=False),
        scratch_types=[
            pltpu.VMEM((V * W,), jnp.int32),
            pltpu.VMEM((K, Q), jnp.int32),
            pltpu.VMEM((K, Q), jnp.int32),
            pltpu.VMEM((E, Q), jnp.float32),
            pltpu.VMEM((E, Q), jnp.float32),
            pltpu.SemaphoreType.DMA,
            pltpu.SemaphoreType.DMA,
            pltpu.SemaphoreType.DMA,
            pltpu.SemaphoreType.DMA,
        ],
    )
    def k(x_hbm, table_hbm, out_hbm, table_v,
          idx_v0, idx_v1, out_v0, out_v1, si0, si1, so0, so1):
        c = lax.axis_index("c")
        s = lax.axis_index("s")
        wid = s * NC + c
        pltpu.sync_copy(table_hbm, table_v)

        idx_bufs, out_bufs = [idx_v0, idx_v1], [out_v0, out_v1]
        sin, sout = [si0, si1], [so0, so1]

        def src(it):
            g = wid * IPW + it
            return g // (P // Q), (g % (P // Q)) * Q

        def compute(idx_v, out_v):
            iota = lax.broadcasted_iota(jnp.int32, (L,), 0)

            def pv_body(pv, carry2):
                rows = [idx_v[kk, pl.ds(pv * L, L)] * W for kk in range(K)]
                p_idx = pv * L + iota

                def gathers(group):
                    j, h = group & 15, group >> 4
                    ep = ((iota + j) & 15) + 16 * h
                    return ep, [plsc.load_gather(table_v, [rows[kk] + ep])
                                for kk in range(K)]

                def reduce_scatter(ep, g):
                    b = [plsc.bitcast(gi, jnp.bfloat16) for gi in g]
                    acc = (((b[0] + b[1]) + (b[2] + b[3]))
                           + (b[4] + b[5]))
                    lo, hi = plsc.unpack(
                        acc, format=plsc.PackFormat.INTERLEAVED)
                    e_lo = 2 * ep
                    plsc.store_scatter(out_v, [e_lo, p_idx], lo)
                    plsc.store_scatter(out_v, [e_lo + 1, p_idx], hi)

                ep, g = gathers(0)
                for group in range(1, EP):
                    nep, ng = gathers(group)
                    reduce_scatter(ep, g)
                    ep, g = nep, ng
                reduce_scatter(ep, g)
                return carry2

            lax.fori_loop(0, Q // L, pv_body, 0)

        bt0, p00 = src(0)
        hin = [pltpu.async_copy(
            x_hbm.at[bt0, :, pl.ds(p00, Q)], idx_bufs[0], sin[0]), None]
        hout = [None, None]
        for it in range(IPW):
            b = it & 1
            if it + 1 < IPW:
                btn, p0n = src(it + 1)
                hin[b ^ 1] = pltpu.async_copy(
                    x_hbm.at[btn, :, pl.ds(p0n, Q)], idx_bufs[b ^ 1],
                    sin[b ^ 1])
            hin[b].wait()
            if hout[b] is not None:
                hout[b].wait()
            compute(idx_bufs[b], out_bufs[b])
            bt, p0 = src(it)
            hout[b] = pltpu.async_copy(
                out_bufs[b], out_hbm.at[bt, :, pl.ds(p0, Q)], sout[b])
        hout[0].wait()
        hout[1].wait()

    return k(x2, tpack)


def kernel(x, table):
    x2 = x.reshape(BT, K, P)
    tpack = jax.lax.bitcast_convert_type(
        table.astype(jnp.bfloat16).reshape(V, W, 2), jnp.int32).reshape(V * W)
    out = _sc_embed(x2, tpack)
    return out.reshape(16, 8, E, 32, 32)
